# Initial kernel scaffold; baseline (speedup 1.0000x reference)
import jax
import jax.numpy as jnp
from jax.experimental import pallas as pl


def kernel(src_ids, dst_ids, interact_times, static_node_feats, nbr_nids, nbr_times, nbr_feats, time_w, time_b, Wq, Wk, Wv, Wo, fc1_w, fc1_b, fc2_w, fc2_b):
    raise NotImplementedError("write your pallas kernel here")


# speedup vs baseline: 1.4019x; 1.4019x over previous
import functools
import math

import jax
import jax.numpy as jnp
from jax import lax
from jax.experimental import pallas as pl
from jax.experimental.pallas import tpu as pltpu
from jax.experimental.pallas import tpu_sc as plsc

B = 4096
NBR = 32
ND = 128
ED = 16
TD = 100
EMBED = 128
H = 2
QD = ND + TD
KD = ND + ED + TD
DH = QD // H

B2 = 2 * B
_R = B2 * NBR + B2
_CHUNK = 128
_NCHUNKS = _R // _CHUNK
_NW = 32
_CPW = _NCHUNKS // _NW

BB = 128
_G = B2 // BB

_SCALE = 1.0 / math.sqrt(float(DH))


def _gather_rows(table, idx2d):
    mesh = plsc.VectorSubcoreMesh(core_axis_name="c", subcore_axis_name="s")

    @functools.partial(
        pl.kernel,
        mesh=mesh,
        out_type=jax.ShapeDtypeStruct((_R, ND), jnp.float32),
        scratch_types=[
            pltpu.VMEM((_CPW, _CHUNK), jnp.int32),
            pltpu.VMEM((_CHUNK, ND), jnp.float32),
            pltpu.VMEM((_CHUNK, ND), jnp.float32),
            pltpu.SemaphoreType.DMA,
            pltpu.SemaphoreType.DMA,
        ],
    )
    def k(table_hbm, idx_hbm, out_hbm, idx_v, buf0, buf1, sem0, sem1):
        wid = lax.axis_index("s") * 2 + lax.axis_index("c")
        base = wid * _CPW
        pltpu.sync_copy(idx_hbm.at[wid], idx_v)

        def start(j, buf, sem):
            pltpu.make_async_copy(table_hbm.at[idx_v.at[j]], buf, sem).start()

        def finish(j, buf, sem):
            pltpu.make_async_copy(table_hbm.at[idx_v.at[j]], buf, sem).wait()
            pltpu.sync_copy(buf, out_hbm.at[pl.ds((base + j) * _CHUNK, _CHUNK)])

        start(0, buf0, sem0)
        start(1, buf1, sem1)

        def body(g, carry):
            j0 = 2 * g
            finish(j0, buf0, sem0)

            @pl.when(j0 + 2 < _CPW)
            def _():
                start(j0 + 2, buf0, sem0)

            finish(j0 + 1, buf1, sem1)

            @pl.when(j0 + 3 < _CPW)
            def _():
                start(j0 + 3, buf1, sem1)

            return carry

        lax.fori_loop(0, _CPW // 2, body, 0)

    return k(table, idx2d)


def _dot(a, b):
    return lax.dot_general(a, b, (((1,), (0,)), ((), ())),
                           preferred_element_type=jnp.float32)


def _tc_body(node_ref, nbr_ref, dt_ref, ids_ref, ef_ref, tw_ref, tb_ref,
             wqn_ref, wqt_ref, wkn_ref, wke_ref, wkt_ref,
             wvn_ref, wve_ref, wvt_ref, wo_ref,
             f1a_ref, f1n_ref, b1_ref, f2_ref, b2_ref, out_ref):
    node = node_ref[...]
    nbr = nbr_ref[...].reshape(BB * NBR, ND)
    ef = ef_ref[...].reshape(BB * NBR, ED)
    tw = tw_ref[...]
    tb = tb_ref[...]

    tfeat3 = jnp.cos(dt_ref[...] * tw + tb)
    tfeat = tfeat3.reshape(BB * NBR, TD)
    q_time = jnp.cos(tb).reshape(1, TD)

    Q = _dot(node, wqn_ref[...]) + _dot(q_time, wqt_ref[...])
    K = (_dot(nbr, wkn_ref[...]) + _dot(ef, wke_ref[...])
         + _dot(tfeat, wkt_ref[...]))
    V = (_dot(nbr, wvn_ref[...]) + _dot(ef, wve_ref[...])
         + _dot(tfeat, wvt_ref[...]))

    K3 = K.reshape(BB, NBR, QD)
    P = Q[:, None, :] * K3
    lane = lax.broadcasted_iota(jnp.int32, (1, 1, QD), 2)
    head0 = (lane < DH).astype(jnp.float32)
    s_all = jnp.sum(P, axis=2)
    s0 = jnp.sum(P * head0, axis=2)
    s1 = s_all - s0

    pad = (ids_ref[...] == 0).astype(jnp.float32) * 1e9
    s0 = s0 * _SCALE - pad
    s1 = s1 * _SCALE - pad

    def softmax(s):
        m = jnp.max(s, axis=1, keepdims=True)
        e = jnp.exp(s - m)
        return e / jnp.sum(e, axis=1, keepdims=True)

    a0 = softmax(s0)
    a1 = softmax(s1)
    A = jnp.where(lane < DH, a0[:, :, None], a1[:, :, None])
    attn = jnp.sum(A * V.reshape(BB, NBR, QD), axis=1)

    out = _dot(attn, wo_ref[...])
    h = jnp.maximum(_dot(out, f1a_ref[...]) + _dot(node, f1n_ref[...])
                    + b1_ref[...], 0.0)
    out_ref[...] = _dot(h, f2_ref[...]) + b2_ref[...]


def _full_spec(shape):
    n = len(shape)
    return pl.BlockSpec(shape, lambda i, _n=n: (0,) * _n)


def _tc_grid_spec():
    in_specs = [
        pl.BlockSpec((BB, ND), lambda i: (i, 0)),
        pl.BlockSpec((BB, NBR, ND), lambda i: (i, 0, 0)),
        pl.BlockSpec((BB, NBR, 1), lambda i: (i, 0, 0)),
        pl.BlockSpec((BB, NBR), lambda i: (i, 0)),
        pl.BlockSpec((BB, NBR, ED), lambda i: (i, 0, 0)),
        _full_spec((1, 1, TD)),
        _full_spec((1, 1, TD)),
        _full_spec((ND, QD)),
        _full_spec((TD, QD)),
        _full_spec((ND, QD)),
        _full_spec((ED, QD)),
        _full_spec((TD, QD)),
        _full_spec((ND, QD)),
        _full_spec((ED, QD)),
        _full_spec((TD, QD)),
        _full_spec((QD, QD)),
        _full_spec((QD, EMBED)),
        _full_spec((ND, EMBED)),
        _full_spec((1, EMBED)),
        _full_spec((EMBED, EMBED)),
        _full_spec((1, EMBED)),
    ]
    out_specs = pl.BlockSpec((BB, EMBED), lambda i: (i, 0))
    out_shape = jax.ShapeDtypeStruct((B2, EMBED), jnp.float32)
    return (_G,), in_specs, out_specs, out_shape


def _gather_ids(src_ids, dst_ids, nbr_nids):
    ids_all = jnp.concatenate([
        nbr_nids[:B2].reshape(-1),
        src_ids,
        dst_ids,
    ]).astype(jnp.int32)
    return ids_all


def _tc_inputs(gathered, interact_times, nbr_nids, nbr_times, nbr_feats,
               time_w, time_b, Wq, Wk, Wv, Wo, fc1_w, fc1_b, fc2_w, fc2_b):
    nbr_feat = gathered[:B2 * NBR].reshape(B2, NBR, ND)
    node_raw = gathered[B2 * NBR:]
    t2 = jnp.concatenate([interact_times, interact_times])
    dt = (t2[:, None] - nbr_times[:B2])[:, :, None]
    return (
        node_raw, nbr_feat, dt, nbr_nids[:B2], nbr_feats[:B2],
        time_w.reshape(1, 1, TD), time_b.reshape(1, 1, TD),
        Wq[:ND], Wq[ND:],
        Wk[:ND], Wk[ND:ND + ED], Wk[ND + ED:],
        Wv[:ND], Wv[ND:ND + ED], Wv[ND + ED:],
        Wo,
        fc1_w[:QD], fc1_w[QD:], fc1_b.reshape(1, EMBED),
        fc2_w, fc2_b.reshape(1, EMBED),
    )


def kernel(src_ids, dst_ids, interact_times, static_node_feats, nbr_nids,
           nbr_times, nbr_feats, time_w, time_b, Wq, Wk, Wv, Wo,
           fc1_w, fc1_b, fc2_w, fc2_b):
    ids_all = _gather_ids(src_ids, dst_ids, nbr_nids)
    gathered = _gather_rows(static_node_feats,
                            ids_all.reshape(_NW, _CPW, _CHUNK))
    ops = _tc_inputs(gathered, interact_times, nbr_nids, nbr_times, nbr_feats,
                     time_w, time_b, Wq, Wk, Wv, Wo, fc1_w, fc1_b, fc2_w, fc2_b)
    grid, in_specs, out_specs, out_shape = _tc_grid_spec()
    z = pl.pallas_call(
        _tc_body,
        grid=grid,
        in_specs=in_specs,
        out_specs=out_specs,
        out_shape=out_shape,
    )(*ops)
    return z[:B], z[B:]


# speedup vs baseline: 2.6341x; 1.8789x over previous
import functools
import math

import jax
import jax.numpy as jnp
from jax import lax
from jax.experimental import pallas as pl
from jax.experimental.pallas import tpu as pltpu
from jax.experimental.pallas import tpu_sc as plsc

B = 4096
NBR = 32
ND = 128
ED = 16
TD = 100
EMBED = 128
H = 2
QD = ND + TD
KD = ND + ED + TD
DH = QD // H

_RS = B * NBR + B
_CHUNK = 128
_NCH_S = _RS // _CHUNK
_NW = 32
_CPW = _NCH_S // _NW

BB = 256
_G = B // BB

_SCALE = 1.0 / math.sqrt(float(DH))


def _gather_rows(table, idx3d):
    mesh = plsc.VectorSubcoreMesh(core_axis_name="c", subcore_axis_name="s")

    @functools.partial(
        pl.kernel,
        mesh=mesh,
        out_type=jax.ShapeDtypeStruct((_RS, ND), jnp.float32),
        scratch_types=[
            pltpu.VMEM((_CPW, _CHUNK), jnp.int32),
            pltpu.VMEM((_CHUNK, ND), jnp.float32),
            pltpu.VMEM((_CHUNK, ND), jnp.float32),
            pltpu.SemaphoreType.DMA,
            pltpu.SemaphoreType.DMA,
        ],
    )
    def k(table_hbm, idx_hbm, out_hbm, idx_v, buf0, buf1, sem0, sem1):
        wid = lax.axis_index("s") * 2 + lax.axis_index("c")
        base = wid * _CPW
        pltpu.sync_copy(idx_hbm.at[wid], idx_v)

        def start(j, buf, sem):
            pltpu.make_async_copy(table_hbm.at[idx_v.at[j]], buf, sem).start()

        def finish(j, buf, sem):
            pltpu.make_async_copy(table_hbm.at[idx_v.at[j]], buf, sem).wait()
            pltpu.sync_copy(buf, out_hbm.at[pl.ds((base + j) * _CHUNK, _CHUNK)])

        start(0, buf0, sem0)
        start(1, buf1, sem1)

        def body(g, carry):
            j0 = 2 * g
            finish(j0, buf0, sem0)

            @pl.when(j0 + 2 < _CPW)
            def _():
                start(j0 + 2, buf0, sem0)

            finish(j0 + 1, buf1, sem1)

            @pl.when(j0 + 3 < _CPW)
            def _():
                start(j0 + 3, buf1, sem1)

            return carry

        lax.fori_loop(0, _CPW // 2, body, 0)
        if _CPW % 2 == 1:
            finish(_CPW - 1, buf0, sem0)

    return k(table, idx3d)


def _dot(a, b):
    return lax.dot_general(a, b, (((1,), (0,)), ((), ())),
                           preferred_element_type=jnp.float32)


_TWO_PI = 6.283185307179586
_INV_2PI = 1.0 / _TWO_PI
_COS_C = (-1.1470745597729725e-11, 2.08767569878681e-09,
          -2.755731922398589e-07, 2.48015873015873e-05,
          -0.001388888888888889, 0.041666666666666664, -0.5)


def _cos_poly(x):
    n = jnp.round(x * _INV_2PI)
    t = x - n * _TWO_PI
    u = t * t
    p = jnp.float32(_COS_C[0])
    for c in _COS_C[1:]:
        p = p * u + jnp.float32(c)
    return p * u + 1.0


def _tc_body(node_ref, nbr_ref, dt_ref, ids_ref, ef_ref, tw_ref, tb_ref,
             wqn_ref, wqt_ref, wkn_ref, wke_ref, wkt_ref,
             wvn_ref, wve_ref, wvt_ref, wo_ref,
             f1a_ref, f1n_ref, b1_ref, f2_ref, b2_ref, out_ref):
    node = node_ref[...]
    nbr = nbr_ref[...].reshape(BB * NBR, ND)
    ef = ef_ref[...].reshape(BB * NBR, ED)
    tw = tw_ref[...]
    tb = tb_ref[...]

    tfeat3 = _cos_poly(dt_ref[...] * tw + tb)
    tfeat = tfeat3.reshape(BB * NBR, TD)
    q_time = jnp.cos(tb).reshape(1, TD)

    bf = jnp.bfloat16
    nbr16 = nbr.astype(bf)
    ef16 = ef.astype(bf)
    tf16 = tfeat.astype(bf)
    Q = _dot(node, wqn_ref[...]) + _dot(q_time, wqt_ref[...])
    K = (_dot(nbr16, wkn_ref[...]) + _dot(ef16, wke_ref[...])
         + _dot(tf16, wkt_ref[...]))
    V = (_dot(nbr16, wvn_ref[...]) + _dot(ef16, wve_ref[...])
         + _dot(tf16, wvt_ref[...]))

    K3 = K.reshape(BB, NBR, QD)
    P16 = (Q[:, None, :] * K3).astype(bf).reshape(BB * NBR, QD)
    row = lax.broadcasted_iota(jnp.int32, (QD, QD), 0)
    w0 = (row < DH).astype(bf)
    w1 = (row >= DH).astype(bf)
    s03 = _dot(P16, w0).reshape(BB, NBR, QD)
    s13 = _dot(P16, w1).reshape(BB, NBR, QD)

    pad = (ids_ref[...] == 0).astype(jnp.float32) * 1e9
    s03 = s03 * _SCALE - pad
    s13 = s13 * _SCALE - pad

    def softmax(s):
        m = jnp.max(s, axis=1, keepdims=True)
        e = jnp.exp(s - m)
        return e / jnp.sum(e, axis=1, keepdims=True)

    a0 = softmax(s03)
    a1 = softmax(s13)
    lane = lax.broadcasted_iota(jnp.int32, (1, 1, QD), 2)
    A = jnp.where(lane < DH, a0, a1)
    attn = jnp.sum(A * V.reshape(BB, NBR, QD), axis=1)

    out = _dot(attn, wo_ref[...])
    h = jnp.maximum(_dot(out, f1a_ref[...]) + _dot(node, f1n_ref[...])
                    + b1_ref[...], 0.0)
    out_ref[...] = _dot(h, f2_ref[...]) + b2_ref[...]


def _full_spec(shape):
    n = len(shape)
    return pl.BlockSpec(shape, lambda i, _n=n: (0,) * _n)


def _tc_grid_spec():
    in_specs = [
        pl.BlockSpec((BB, ND), lambda i: (i, 0)),
        pl.BlockSpec((BB, NBR, ND), lambda i: (i, 0, 0)),
        pl.BlockSpec((BB, NBR, 1), lambda i: (i, 0, 0)),
        pl.BlockSpec((BB, NBR, 1), lambda i: (i, 0, 0)),
        pl.BlockSpec((BB, NBR, ED), lambda i: (i, 0, 0)),
        _full_spec((1, 1, TD)),
        _full_spec((1, 1, TD)),
        _full_spec((ND, QD)),
        _full_spec((TD, QD)),
        _full_spec((ND, QD)),
        _full_spec((ED, QD)),
        _full_spec((TD, QD)),
        _full_spec((ND, QD)),
        _full_spec((ED, QD)),
        _full_spec((TD, QD)),
        _full_spec((QD, QD)),
        _full_spec((QD, EMBED)),
        _full_spec((ND, EMBED)),
        _full_spec((1, EMBED)),
        _full_spec((EMBED, EMBED)),
        _full_spec((1, EMBED)),
    ]
    out_specs = pl.BlockSpec((BB, EMBED), lambda i: (i, 0))
    out_shape = jax.ShapeDtypeStruct((B, EMBED), jnp.float32)
    return (_G,), in_specs, out_specs, out_shape


def _gather_ids(node_ids, nbr_nids_side):
    return jnp.concatenate([
        nbr_nids_side.reshape(-1),
        node_ids,
    ]).astype(jnp.int32)


def _tc_inputs(gathered, interact_times, nbr_nids_s, nbr_times_s, nbr_feats_s,
               time_w, time_b, Wq, Wk, Wv, Wo, fc1_w, fc1_b, fc2_w, fc2_b):
    nbr_feat = gathered[:B * NBR].reshape(B, NBR, ND)
    node_raw = gathered[B * NBR:]
    dt = (interact_times[:, None] - nbr_times_s)[:, :, None]
    return (
        node_raw, nbr_feat, dt, nbr_nids_s[:, :, None], nbr_feats_s,
        time_w.reshape(1, 1, TD), time_b.reshape(1, 1, TD),
        Wq[:ND], Wq[ND:],
        Wk[:ND].astype(jnp.bfloat16), Wk[ND:ND + ED].astype(jnp.bfloat16),
        Wk[ND + ED:].astype(jnp.bfloat16),
        Wv[:ND].astype(jnp.bfloat16), Wv[ND:ND + ED].astype(jnp.bfloat16),
        Wv[ND + ED:].astype(jnp.bfloat16),
        Wo,
        fc1_w[:QD], fc1_w[QD:], fc1_b.reshape(1, EMBED),
        fc2_w, fc2_b.reshape(1, EMBED),
    )


def kernel(src_ids, dst_ids, interact_times, static_node_feats, nbr_nids,
           nbr_times, nbr_feats, time_w, time_b, Wq, Wk, Wv, Wo,
           fc1_w, fc1_b, fc2_w, fc2_b):
    grid, in_specs, out_specs, out_shape = _tc_grid_spec()
    tc_call = pl.pallas_call(
        _tc_body,
        grid=grid,
        in_specs=in_specs,
        out_specs=out_specs,
        out_shape=out_shape,
    )
    zs = []
    for node_ids, lo in ((src_ids, 0), (dst_ids, B)):
        ids_side = _gather_ids(node_ids, nbr_nids[lo:lo + B])
        gathered = _gather_rows(static_node_feats,
                                ids_side.reshape(_NW, _CPW, _CHUNK))
        ops = _tc_inputs(gathered, interact_times, nbr_nids[lo:lo + B],
                         nbr_times[lo:lo + B], nbr_feats[lo:lo + B],
                         time_w, time_b, Wq, Wk, Wv, Wo,
                         fc1_w, fc1_b, fc2_w, fc2_b)
        zs.append((gathered, ops))
    z_out = [tc_call(*ops) for _, ops in zs]
    return z_out[0], z_out[1]


# speedup vs baseline: 3.2851x; 1.2472x over previous
import functools
import math

import jax
import jax.numpy as jnp
from jax import lax
from jax.experimental import pallas as pl
from jax.experimental.pallas import tpu as pltpu
from jax.experimental.pallas import tpu_sc as plsc

B = 4096
NBR = 32
ND = 128
ED = 16
TD = 100
EMBED = 128
H = 2
QD = ND + TD
KD = ND + ED + TD
DH = QD // H

_RS = B * NBR + B
_CHUNK = 128
_NCH_S = _RS // _CHUNK
_NW = 32
_CPW = _NCH_S // _NW

BB = 128
_G = B // BB

_SCALE = 1.0 / math.sqrt(float(DH))


_NBR_CPW = (B * NBR) // _CHUNK // _NW
assert _CPW == _NBR_CPW + 1


def _gather_rows(table, idx3d):
    mesh = plsc.VectorSubcoreMesh(core_axis_name="c", subcore_axis_name="s")

    @functools.partial(
        pl.kernel,
        mesh=mesh,
        out_type=(jax.ShapeDtypeStruct((B * NBR, ND), jnp.float32),
                  jax.ShapeDtypeStruct((B, ND), jnp.float32)),
        scratch_types=[
            pltpu.VMEM((_CPW, _CHUNK), jnp.int32),
            pltpu.VMEM((_CHUNK, ND), jnp.float32),
            pltpu.VMEM((_CHUNK, ND), jnp.float32),
            pltpu.VMEM((_CHUNK, ND), jnp.float32),
            pltpu.VMEM((_CHUNK, ND), jnp.float32),
            pltpu.SemaphoreType.DMA,
            pltpu.SemaphoreType.DMA,
            pltpu.SemaphoreType.DMA,
            pltpu.SemaphoreType.DMA,
            pltpu.SemaphoreType.DMA,
            pltpu.SemaphoreType.DMA,
            pltpu.SemaphoreType.DMA,
            pltpu.SemaphoreType.DMA,
        ],
    )
    def k(table_hbm, idx_hbm, nbr_hbm, node_hbm, idx_v,
          b0, b1, b2, b3, g0, g1, g2, g3, w0, w1, w2, w3):
        bufs = (b0, b1, b2, b3)
        gsems = (g0, g1, g2, g3)
        wsems = (w0, w1, w2, w3)
        wid = lax.axis_index("s") * 2 + lax.axis_index("c")
        pltpu.sync_copy(idx_hbm.at[wid], idx_v)

        def start_gather(j, b):
            pltpu.make_async_copy(table_hbm.at[idx_v.at[j]], bufs[b],
                                  gsems[b]).start()

        def wait_gather(j, b):
            pltpu.make_async_copy(table_hbm.at[idx_v.at[j]], bufs[b],
                                  gsems[b]).wait()

        def start_write(j, b):
            @pl.when(j < _NBR_CPW)
            def _():
                pltpu.make_async_copy(
                    bufs[b],
                    nbr_hbm.at[pl.ds((wid * _NBR_CPW + j) * _CHUNK, _CHUNK)],
                    wsems[b]).start()

            @pl.when(j == _NBR_CPW)
            def _():
                pltpu.make_async_copy(
                    bufs[b], node_hbm.at[pl.ds(wid * _CHUNK, _CHUNK)],
                    wsems[b]).start()

        def drain_write(b):
            pltpu.make_async_copy(
                bufs[b], nbr_hbm.at[pl.ds(0, _CHUNK)], wsems[b]).wait()

        def unit(j, b):
            wait_gather(j, b)
            start_write(j, b)
            nb = (b + 2) % 4

            @pl.when((j + 2 < _CPW) & (j >= 2))
            def _():
                drain_write(nb)

            @pl.when(j + 2 < _CPW)
            def _():
                start_gather(j + 2, nb)

        start_gather(0, 0)
        start_gather(1, 1)

        def body(g, carry):
            j0 = 4 * g
            for b in range(4):
                unit(j0 + b, b)
            return carry

        lax.fori_loop(0, (_CPW - 1) // 4, body, 0)
        for j in range(_CPW - 1 - (_CPW - 1) % 4, _CPW):
            unit(j, j % 4)
        for b in range(4):
            drain_write(b)

    return k(table, idx3d)


def _dot(a, b):
    return lax.dot_general(a, b, (((1,), (0,)), ((), ())),
                           preferred_element_type=jnp.float32)


_TWO_PI = 6.283185307179586
_INV_2PI = 1.0 / _TWO_PI
_COS_C = (-2.2063688697696682e-07, 2.4226569073610363e-05,
          -0.0013860990081225792, 0.04166064716416637,
          -0.4999955256402951, 0.9999996603166155)


def _cos_poly(x):
    n = jnp.round(x * _INV_2PI)
    t = x - n * _TWO_PI
    u = t * t
    p = jnp.float32(_COS_C[0])
    for c in _COS_C[1:]:
        p = p * u + jnp.float32(c)
    return p


def _tc_body(node_ref, nbr_ref, dt_ref, ids_ref, ef_ref, tw_ref, tb_ref,
             wqn_ref, wqt_ref, wkn_ref, wke_ref, wkt_ref,
             wvn_ref, wve_ref, wvt_ref, wo_ref,
             f1a_ref, f1n_ref, b1_ref, f2_ref, b2_ref, out_ref):
    node = node_ref[...]
    nbr = nbr_ref[...].reshape(BB * NBR, ND)
    ef = ef_ref[...].reshape(BB * NBR, ED)
    tw = tw_ref[...]
    tb = tb_ref[...]

    tfeat3 = _cos_poly(dt_ref[...] * tw + tb)
    tfeat = tfeat3.reshape(BB * NBR, TD)
    q_time = jnp.cos(tb).reshape(1, TD)

    bf = jnp.bfloat16
    nbr16 = nbr.astype(bf)
    ef16 = ef.astype(bf)
    tf16 = tfeat.astype(bf)
    Q = _dot(node, wqn_ref[...]) + _dot(q_time, wqt_ref[...])
    K = (_dot(nbr16, wkn_ref[...]) + _dot(ef16, wke_ref[...])
         + _dot(tf16, wkt_ref[...]))
    V = (_dot(nbr16, wvn_ref[...]) + _dot(ef16, wve_ref[...])
         + _dot(tf16, wvt_ref[...]))

    K3 = K.reshape(BB, NBR, QD)
    P16 = (Q[:, None, :] * K3).astype(bf).reshape(BB * NBR, QD)
    row = lax.broadcasted_iota(jnp.int32, (QD, QD), 0)
    col = lax.broadcasted_iota(jnp.int32, (QD, QD), 1)
    w_sel = ((row < DH) == (col < DH)).astype(bf)
    S3 = _dot(P16, w_sel).reshape(BB, NBR, QD)

    pad = (ids_ref[...] == 0).astype(jnp.float32) * 1e9
    S3 = S3 * _SCALE - pad

    m = jnp.max(S3, axis=1, keepdims=True)
    e = jnp.exp(S3 - m)
    denom = jnp.sum(e, axis=1)
    attn = jnp.sum(e * V.reshape(BB, NBR, QD), axis=1) / denom

    out = _dot(attn, wo_ref[...])
    h = jnp.maximum(_dot(out, f1a_ref[...]) + _dot(node, f1n_ref[...])
                    + b1_ref[...], 0.0)
    out_ref[...] = _dot(h, f2_ref[...]) + b2_ref[...]


def _full_spec(shape):
    n = len(shape)
    return pl.BlockSpec(shape, lambda i, _n=n: (0,) * _n)


def _tc_grid_spec():
    in_specs = [
        pl.BlockSpec((BB, ND), lambda i: (i, 0)),
        pl.BlockSpec((BB, NBR, ND), lambda i: (i, 0, 0)),
        pl.BlockSpec((BB, NBR, 1), lambda i: (i, 0, 0)),
        pl.BlockSpec((BB, NBR, 1), lambda i: (i, 0, 0)),
        pl.BlockSpec((BB, NBR, ED), lambda i: (i, 0, 0)),
        _full_spec((1, 1, TD)),
        _full_spec((1, 1, TD)),
        _full_spec((ND, QD)),
        _full_spec((TD, QD)),
        _full_spec((ND, QD)),
        _full_spec((ED, QD)),
        _full_spec((TD, QD)),
        _full_spec((ND, QD)),
        _full_spec((ED, QD)),
        _full_spec((TD, QD)),
        _full_spec((QD, QD)),
        _full_spec((QD, EMBED)),
        _full_spec((ND, EMBED)),
        _full_spec((1, EMBED)),
        _full_spec((EMBED, EMBED)),
        _full_spec((1, EMBED)),
    ]
    out_specs = pl.BlockSpec((BB, EMBED), lambda i: (i, 0))
    out_shape = jax.ShapeDtypeStruct((B, EMBED), jnp.float32)
    return (_G,), in_specs, out_specs, out_shape


def _gather_ids(node_ids, nbr_nids_side):
    nbr2 = nbr_nids_side.reshape(_NW, _NBR_CPW * _CHUNK)
    node2 = node_ids.reshape(_NW, _CHUNK)
    return jnp.concatenate([nbr2, node2], axis=1).astype(jnp.int32) \
              .reshape(_NW, _CPW, _CHUNK)


def _tc_inputs(nbr_rows, node_rows, interact_times, nbr_nids_s, nbr_times_s,
               nbr_feats_s, time_w, time_b, Wq, Wk, Wv, Wo,
               fc1_w, fc1_b, fc2_w, fc2_b):
    nbr_feat = nbr_rows.reshape(B, NBR, ND)
    node_raw = node_rows
    dt = (interact_times[:, None] - nbr_times_s)[:, :, None]
    return (
        node_raw, nbr_feat, dt, nbr_nids_s[:, :, None], nbr_feats_s,
        time_w.reshape(1, 1, TD), time_b.reshape(1, 1, TD),
        Wq[:ND], Wq[ND:],
        Wk[:ND].astype(jnp.bfloat16), Wk[ND:ND + ED].astype(jnp.bfloat16),
        Wk[ND + ED:].astype(jnp.bfloat16),
        Wv[:ND].astype(jnp.bfloat16), Wv[ND:ND + ED].astype(jnp.bfloat16),
        Wv[ND + ED:].astype(jnp.bfloat16),
        Wo,
        fc1_w[:QD], fc1_w[QD:], fc1_b.reshape(1, EMBED),
        fc2_w, fc2_b.reshape(1, EMBED),
    )


def kernel(src_ids, dst_ids, interact_times, static_node_feats, nbr_nids,
           nbr_times, nbr_feats, time_w, time_b, Wq, Wk, Wv, Wo,
           fc1_w, fc1_b, fc2_w, fc2_b):
    grid, in_specs, out_specs, out_shape = _tc_grid_spec()
    tc_call = pl.pallas_call(
        _tc_body,
        grid=grid,
        in_specs=in_specs,
        out_specs=out_specs,
        out_shape=out_shape,
    )
    zs = []
    for node_ids, lo in ((src_ids, 0), (dst_ids, B)):
        ids_side = _gather_ids(node_ids, nbr_nids[lo:lo + B])
        nbr_rows, node_rows = _gather_rows(static_node_feats, ids_side)
        ops = _tc_inputs(nbr_rows, node_rows, interact_times,
                         nbr_nids[lo:lo + B], nbr_times[lo:lo + B],
                         nbr_feats[lo:lo + B], time_w, time_b,
                         Wq, Wk, Wv, Wo, fc1_w, fc1_b, fc2_w, fc2_b)
        zs.append(ops)
    z_out = [tc_call(*ops) for ops in zs]
    return z_out[0], z_out[1]
